# baseline (speedup 1.0000x reference)
import functools

import jax
import jax.numpy as jnp
from jax import lax
from jax.experimental import pallas as pl
from jax.experimental.pallas import tpu as pltpu
from jax.experimental.pallas import tpu_sc as plsc

NC = 2
NS = 16
L = 16
NW = NC * NS

BATCH = 16384
D = 32
B_PER_W = BATCH // NW
CHUNK = 128
NCHUNK = B_PER_W // CHUNK


def _gmf_sc(uid3, iid3, user_table, item_table):
    mesh = plsc.VectorSubcoreMesh(core_axis_name="c", subcore_axis_name="s")

    @functools.partial(
        pl.kernel,
        mesh=mesh,
        out_type=jax.ShapeDtypeStruct((BATCH, D), jnp.float32),
        compiler_params=pltpu.CompilerParams(use_tc_tiling_on_sc=False),
        scratch_types=[
            pltpu.VMEM((NCHUNK, CHUNK), jnp.int32),
            pltpu.VMEM((NCHUNK, CHUNK), jnp.int32),
            pltpu.VMEM((B_PER_W, D), jnp.float32),
            pltpu.VMEM((B_PER_W, D), jnp.float32),
            pltpu.SemaphoreType.DMA,
        ],
    )
    def k(uid_hbm, iid_hbm, ut_hbm, it_hbm, out_hbm, uidx, iidx, urows, irows, sem):
        wid = lax.axis_index("s") * NC + lax.axis_index("c")
        base = wid * B_PER_W

        pltpu.sync_copy(uid_hbm.at[wid], uidx)
        pltpu.sync_copy(iid_hbm.at[wid], iidx)

        copies = []
        for j in range(NCHUNK):
            copies.append(pltpu.async_copy(
                ut_hbm.at[uidx.at[j]], urows.at[pl.ds(j * CHUNK, CHUNK)], sem))
            copies.append(pltpu.async_copy(
                it_hbm.at[iidx.at[j]], irows.at[pl.ds(j * CHUNK, CHUNK)], sem))
        for c in copies:
            c.wait()

        def body(r, carry):
            for c in range(D // L):
                sl = pl.ds(c * L, L)
                urows[r, sl] = urows[r, sl] * irows[r, sl]
            return carry

        lax.fori_loop(0, B_PER_W, body, 0)

        pltpu.sync_copy(urows, out_hbm.at[pl.ds(base, B_PER_W)])

    return k(uid3, iid3, user_table, item_table)


def kernel(user_ids, item_ids, user_table, item_table):
    uid3 = user_ids.astype(jnp.int32).reshape(NW, NCHUNK, CHUNK)
    iid3 = item_ids.astype(jnp.int32).reshape(NW, NCHUNK, CHUNK)
    return _gmf_sc(uid3, iid3, user_table, item_table)


# speedup vs baseline: 3.2749x; 3.2749x over previous
import functools

import jax
import jax.numpy as jnp
from jax import lax
from jax.experimental import pallas as pl
from jax.experimental.pallas import tpu as pltpu
from jax.experimental.pallas import tpu_sc as plsc

NC = 2
NS = 16
L = 16
NW = NC * NS

BATCH = 16384
D = 32
V = 1_000_000
B_PER_W = BATCH // NW
NSLOT = 8


def _gmf_sc(uid, iid, ut_t, it_t):
    mesh = plsc.VectorSubcoreMesh(core_axis_name="c", subcore_axis_name="s")

    @functools.partial(
        pl.kernel,
        mesh=mesh,
        out_type=jax.ShapeDtypeStruct((D, BATCH), jnp.float32),
        compiler_params=pltpu.CompilerParams(needs_layout_passes=False),
        scratch_types=[
            pltpu.VMEM((B_PER_W,), jnp.int32),
            pltpu.VMEM((B_PER_W,), jnp.int32),
            pltpu.VMEM((NSLOT, D, 128), jnp.float32),
            pltpu.VMEM((NSLOT, D, 128), jnp.float32),
            pltpu.VMEM((D, B_PER_W), jnp.float32),
            pltpu.SemaphoreType.DMA,
            pltpu.SemaphoreType.DMA,
        ],
    )
    def k(uid_hbm, iid_hbm, ut_hbm, it_hbm, out_hbm,
          uloc, iloc, ublk, iblk, obuf, usem, isem):
        wid = lax.axis_index("s") * NC + lax.axis_index("c")
        b0 = wid * B_PER_W

        pltpu.sync_copy(uid_hbm.at[pl.ds(b0, B_PER_W)], uloc)
        pltpu.sync_copy(iid_hbm.at[pl.ds(b0, B_PER_W)], iloc)

        rows_lo = lax.iota(jnp.int32, L)
        rows_hi = rows_lo + L

        def step(g, carry):
            vu = uloc[pl.ds(g * L, L)]
            vi = iloc[pl.ds(g * L, L)]
            tu = vu >> 7
            wu = vu & 127
            ti = vi >> 7
            wi = vi & 127
            for h in range(2):
                ucps, icps = [], []
                for j in range(NSLOT):
                    jj = h * NSLOT + j
                    ucps.append(pltpu.async_copy(
                        ut_hbm.at[:, pl.ds(tu[jj] * 128, 128)], ublk.at[j], usem))
                    icps.append(pltpu.async_copy(
                        it_hbm.at[:, pl.ds(ti[jj] * 128, 128)], iblk.at[j], isem))
                for cp in ucps:
                    cp.wait()
                for cp in icps:
                    cp.wait()
                for j in range(NSLOT):
                    jj = h * NSLOT + j
                    b = g * L + jj
                    bcol = jnp.full((L,), b, jnp.int32)
                    wub = jnp.full((L,), wu[jj], jnp.int32)
                    wib = jnp.full((L,), wi[jj], jnp.int32)
                    for rows in (rows_lo, rows_hi):
                        uv = plsc.load_gather(ublk.at[j], [rows, wub])
                        iv = plsc.load_gather(iblk.at[j], [rows, wib])
                        plsc.store_scatter(obuf, [rows, bcol], uv * iv)
            return carry

        lax.fori_loop(0, B_PER_W // L, step, 0)

        pltpu.sync_copy(obuf, out_hbm.at[:, pl.ds(b0, B_PER_W)])

    return k(uid, iid, ut_t, it_t)


def kernel(user_ids, item_ids, user_table, item_table):
    uid = user_ids.astype(jnp.int32)
    iid = item_ids.astype(jnp.int32)
    out_t = _gmf_sc(uid, iid, user_table.T, item_table.T)
    return out_t.T


# speedup vs baseline: 3.7427x; 1.1428x over previous
import functools

import jax
import jax.numpy as jnp
from jax import lax
from jax.experimental import pallas as pl
from jax.experimental.pallas import tpu as pltpu
from jax.experimental.pallas import tpu_sc as plsc

NC = 2
NS = 16
L = 16
NW = NC * NS

BATCH = 16384
D = 32
V = 1_000_000
B_PER_W = BATCH // NW
NSLOT = 8


def _gmf_sc(uid, iid, ut_t, it_t):
    mesh = plsc.VectorSubcoreMesh(core_axis_name="c", subcore_axis_name="s")

    @functools.partial(
        pl.kernel,
        mesh=mesh,
        out_type=jax.ShapeDtypeStruct((D, BATCH), jnp.float32),
        compiler_params=pltpu.CompilerParams(needs_layout_passes=False),
        scratch_types=[
            pltpu.VMEM((B_PER_W,), jnp.int32),
            pltpu.VMEM((B_PER_W,), jnp.int32),
            pltpu.VMEM((NSLOT, D, 128), jnp.float32),
            pltpu.VMEM((NSLOT, D, 128), jnp.float32),
            pltpu.VMEM((D, B_PER_W), jnp.float32),
            pltpu.SemaphoreType.DMA,
            pltpu.SemaphoreType.DMA,
        ],
    )
    def k(uid_hbm, iid_hbm, ut_hbm, it_hbm, out_hbm,
          uloc, iloc, ublk, iblk, obuf, usem, isem):
        wid = lax.axis_index("s") * NC + lax.axis_index("c")
        b0 = wid * B_PER_W

        pltpu.sync_copy(uid_hbm.at[pl.ds(b0, B_PER_W)], uloc)
        pltpu.sync_copy(iid_hbm.at[pl.ds(b0, B_PER_W)], iloc)

        rows_lo = lax.iota(jnp.int32, L)
        rows_hi = rows_lo + L

        SB = 4

        def step(g, carry):
            vu = uloc[pl.ds(g * L, L)]
            vi = iloc[pl.ds(g * L, L)]
            tu = vu >> 7
            wu = vu & 127
            ti = vi >> 7
            wi = vi & 127
            cps = {}

            def fire(h):
                lst = []
                for j in range(SB):
                    jj = h * SB + j
                    slot = (h & 1) * SB + j
                    lst.append(pltpu.async_copy(
                        ut_hbm.at[:, pl.ds(tu[jj] * 128, 128)],
                        ublk.at[slot], usem))
                    lst.append(pltpu.async_copy(
                        it_hbm.at[:, pl.ds(ti[jj] * 128, 128)],
                        iblk.at[slot], isem))
                cps[h] = lst

            fire(0)
            for h in range(L // SB):
                if h < L // SB - 1:
                    fire(h + 1)
                for cp in cps[h]:
                    cp.wait()
                for j in range(SB):
                    jj = h * SB + j
                    slot = (h & 1) * SB + j
                    b = g * L + jj
                    bcol = jnp.full((L,), b, jnp.int32)
                    wub = jnp.full((L,), wu[jj], jnp.int32)
                    wib = jnp.full((L,), wi[jj], jnp.int32)
                    for rows in (rows_lo, rows_hi):
                        uv = plsc.load_gather(ublk.at[slot], [rows, wub])
                        iv = plsc.load_gather(iblk.at[slot], [rows, wib])
                        plsc.store_scatter(obuf, [rows, bcol], uv * iv)
            return carry

        lax.fori_loop(0, B_PER_W // L, step, 0)

        pltpu.sync_copy(obuf, out_hbm.at[:, pl.ds(b0, B_PER_W)])

    return k(uid, iid, ut_t, it_t)


def kernel(user_ids, item_ids, user_table, item_table):
    uid = user_ids.astype(jnp.int32)
    iid = item_ids.astype(jnp.int32)
    out_t = _gmf_sc(uid, iid, user_table.T, item_table.T)
    return out_t.T


# speedup vs baseline: 3.7687x; 1.0070x over previous
import functools

import jax
import jax.numpy as jnp
from jax import lax
from jax.experimental import pallas as pl
from jax.experimental.pallas import tpu as pltpu
from jax.experimental.pallas import tpu_sc as plsc

NC = 2
NS = 16
L = 16
NW = NC * NS

BATCH = 16384
D = 32
V = 1_000_000
B_PER_W = BATCH // NW
NSLOT = 8


def _gmf_sc(uid, iid, ut_t, it_t):
    mesh = plsc.VectorSubcoreMesh(core_axis_name="c", subcore_axis_name="s")

    @functools.partial(
        pl.kernel,
        mesh=mesh,
        out_type=jax.ShapeDtypeStruct((D, BATCH), jnp.float32),
        compiler_params=pltpu.CompilerParams(needs_layout_passes=False),
        scratch_types=[
            pltpu.VMEM((B_PER_W,), jnp.int32),
            pltpu.VMEM((B_PER_W,), jnp.int32),
            pltpu.VMEM((3 * 4, D, 128), jnp.float32),
            pltpu.VMEM((3 * 4, D, 128), jnp.float32),
            pltpu.VMEM((D, B_PER_W), jnp.float32),
            pltpu.SemaphoreType.DMA,
            pltpu.SemaphoreType.DMA,
        ],
    )
    def k(uid_hbm, iid_hbm, ut_hbm, it_hbm, out_hbm,
          uloc, iloc, ublk, iblk, obuf, usem, isem):
        wid = lax.axis_index("s") * NC + lax.axis_index("c")
        b0 = wid * B_PER_W

        pltpu.sync_copy(uid_hbm.at[pl.ds(b0, B_PER_W)], uloc)
        pltpu.sync_copy(iid_hbm.at[pl.ds(b0, B_PER_W)], iloc)

        rows_lo = lax.iota(jnp.int32, L)
        rows_hi = rows_lo + L

        SB = 4

        def step(g, carry):
            vu = uloc[pl.ds(g * L, L)]
            vi = iloc[pl.ds(g * L, L)]
            tu = vu >> 7
            wu = vu & 127
            ti = vi >> 7
            wi = vi & 127
            cps = {}

            def fire(h):
                lst = []
                for j in range(SB):
                    jj = h * SB + j
                    slot = (h % 3) * SB + j
                    lst.append(pltpu.async_copy(
                        ut_hbm.at[:, pl.ds(tu[jj] * 128, 128)],
                        ublk.at[slot], usem))
                    lst.append(pltpu.async_copy(
                        it_hbm.at[:, pl.ds(ti[jj] * 128, 128)],
                        iblk.at[slot], isem))
                cps[h] = lst

            fire(0)
            fire(1)
            for h in range(L // SB):
                if h < L // SB - 2:
                    fire(h + 2)
                for cp in cps[h]:
                    cp.wait()
                for j in range(SB):
                    jj = h * SB + j
                    slot = (h % 3) * SB + j
                    b = g * L + jj
                    bcol = jnp.full((L,), b, jnp.int32)
                    wub = jnp.full((L,), wu[jj], jnp.int32)
                    wib = jnp.full((L,), wi[jj], jnp.int32)
                    for rows in (rows_lo, rows_hi):
                        uv = plsc.load_gather(ublk.at[slot], [rows, wub])
                        iv = plsc.load_gather(iblk.at[slot], [rows, wib])
                        plsc.store_scatter(obuf, [rows, bcol], uv * iv)
            return carry

        lax.fori_loop(0, B_PER_W // L, step, 0)

        pltpu.sync_copy(obuf, out_hbm.at[:, pl.ds(b0, B_PER_W)])

    return k(uid, iid, ut_t, it_t)


def kernel(user_ids, item_ids, user_table, item_table):
    uid = user_ids.astype(jnp.int32)
    iid = item_ids.astype(jnp.int32)
    out_t = _gmf_sc(uid, iid, user_table.T, item_table.T)
    return out_t.T
